# Initial kernel scaffold; baseline (speedup 1.0000x reference)
import jax
import jax.numpy as jnp
from jax.experimental import pallas as pl


def kernel(x, edge_index, W1, b1, a1, W2, b2, a2):
    raise NotImplementedError("write your pallas kernel here")


# speedup vs baseline: 13.9401x; 13.9401x over previous
import functools

import jax
import jax.numpy as jnp
from jax import lax
from jax.experimental import pallas as pl
from jax.experimental.pallas import tpu as pltpu
from jax.experimental.pallas import tpu_sc as plsc

NC = 2
NS = 16
NW = NC * NS
CHUNK = 128
BLK = 1024


def _cdiv(a, b):
  return (a + b - 1) // b


def _deg_body(n_pad, cpt, dst_hbm, zeros_hbm, parts_hbm, didx_v, ones_v, dacc):
  cid = lax.axis_index("c")
  sid = lax.axis_index("s")
  wid = sid * NC + cid
  rpt = n_pad // NS
  pltpu.sync_copy(zeros_hbm.at[pl.ds(sid * rpt, rpt)],
                  dacc.at[pl.ds(sid * rpt, rpt)])
  pltpu.sync_copy(dst_hbm.at[wid], didx_v)
  for j in range(CHUNK // 16):
    ones_v[pl.ds(j * 16, 16)] = jnp.ones((16,), jnp.float32)
  plsc.subcore_barrier()

  def chunk(i, carry):
    pltpu.sync_copy(ones_v, dacc.at[didx_v.at[i]], add=True)
    return carry

  lax.fori_loop(0, cpt, chunk, 0)
  plsc.subcore_barrier()
  pltpu.sync_copy(dacc.at[pl.ds(sid * rpt, rpt)],
                  parts_hbm.at[cid, pl.ds(sid * rpt, rpt)])


def _agg_body(n_pad, d, cpt, y_hbm, src_hbm, dst_hbm, zeros_hbm, parts_hbm,
              sidx_v, didx_v, rows_v, acc, sem):
  cid = lax.axis_index("c")
  sid = lax.axis_index("s")
  wid = sid * NC + cid
  rpt = n_pad // NS
  pltpu.sync_copy(zeros_hbm.at[pl.ds(sid * rpt, rpt)],
                  acc.at[pl.ds(sid * rpt, rpt)])
  pltpu.sync_copy(src_hbm.at[wid], sidx_v)
  pltpu.sync_copy(dst_hbm.at[wid], didx_v)
  plsc.subcore_barrier()

  def chunk(i, carry):
    pltpu.async_copy(y_hbm.at[sidx_v.at[i]], rows_v, sem).wait()
    pltpu.sync_copy(rows_v, acc.at[didx_v.at[i]], add=True)
    return carry

  lax.fori_loop(0, cpt, chunk, 0)
  plsc.subcore_barrier()
  pltpu.sync_copy(acc.at[pl.ds(sid * rpt, rpt)],
                  parts_hbm.at[cid, pl.ds(sid * rpt, rpt)])


def _sc_mesh():
  return plsc.VectorSubcoreMesh(core_axis_name="c", subcore_axis_name="s",
                                num_cores=NC, num_subcores=NS)


def _deg_kernel(n_pad, cpt):
  return pl.kernel(
      functools.partial(_deg_body, n_pad, cpt),
      out_type=jax.ShapeDtypeStruct((NC, n_pad), jnp.float32),
      mesh=_sc_mesh(),
      scratch_types=[
          pltpu.VMEM((cpt, CHUNK), jnp.int32),
          pltpu.VMEM((CHUNK,), jnp.float32),
          pltpu.VMEM_SHARED((n_pad,), jnp.float32),
      ],
  )


def _agg_kernel(n_pad, d, cpt):
  return pl.kernel(
      functools.partial(_agg_body, n_pad, d, cpt),
      out_type=jax.ShapeDtypeStruct((NC, n_pad, d), jnp.float32),
      mesh=_sc_mesh(),
      scratch_types=[
          pltpu.VMEM((cpt, CHUNK), jnp.int32),
          pltpu.VMEM((cpt, CHUNK), jnp.int32),
          pltpu.VMEM((CHUNK, d), jnp.float32),
          pltpu.VMEM_SHARED((n_pad, d), jnp.float32),
          pltpu.SemaphoreType.DMA,
      ],
  )


def _k1_body(x_ref, w_ref, dp_ref, y_ref, dinv_ref):
  deg = dp_ref[0, :] + dp_ref[1, :] + 1.0
  dinv = lax.rsqrt(deg)
  dinv_ref[...] = dinv
  xw = jnp.dot(x_ref[...], w_ref[...], preferred_element_type=jnp.float32)
  y_ref[...] = xw * dinv[:, None]


def _k2_body(p_ref, y_ref, dinv_ref, w_ref, b_ref, a_ref, o_ref):
  dinv = dinv_ref[...][:, None]
  t = (p_ref[0] + p_ref[1] + y_ref[...]) * dinv + b_ref[...]
  z = jnp.where(t >= 0, t, a_ref[...] * t)
  zw = jnp.dot(z, w_ref[...], preferred_element_type=jnp.float32)
  o_ref[...] = zw * dinv


def _k3_body(p_ref, y_ref, dinv_ref, b_ref, a_ref, o_ref):
  dinv = dinv_ref[...][:, None]
  t = (p_ref[0] + p_ref[1] + y_ref[...]) * dinv + b_ref[...]
  o_ref[...] = jnp.where(t >= 0, t, a_ref[...] * t)


def _row_spec(d):
  return pl.BlockSpec((BLK, d), lambda i: (i, 0))


def _vec_spec():
  return pl.BlockSpec((BLK,), lambda i: (i,))


def _parts_spec(d):
  return pl.BlockSpec((NC, BLK, d), lambda i: (0, i, 0))


def _full_spec(shape, nd):
  return pl.BlockSpec(shape, lambda i: (0,) * nd)


def kernel(x, edge_index, W1, b1, a1, W2, b2, a2):
  n, d = x.shape
  e = edge_index.shape[1]
  n_pad = _cdiv(n + 1, NS * 8) * NS * 8
  n_pad = _cdiv(n_pad, BLK) * BLK
  cpt = _cdiv(e, NW * CHUNK)
  e_pad = cpt * NW * CHUNK
  grid = n_pad // BLK

  ei = edge_index.astype(jnp.int32)
  pad = jnp.full((2, e_pad - e), n, jnp.int32)
  ei = jnp.concatenate([ei, pad], axis=1)
  src = ei[0].reshape(NW, cpt, CHUNK)
  dst = ei[1].reshape(NW, cpt, CHUNK)

  x_pad = jnp.zeros((n_pad, d), x.dtype).at[:n].set(x)
  zeros_1d = jnp.zeros((n_pad,), jnp.float32)
  zeros_2d = jnp.zeros((n_pad, d), jnp.float32)
  b1r, a1r = b1.reshape(1, d), a1.reshape(1, d)
  b2r, a2r = b2.reshape(1, d), a2.reshape(1, d)

  dparts = _deg_kernel(n_pad, cpt)(dst, zeros_1d)

  k1 = pl.pallas_call(
      _k1_body,
      grid=(grid,),
      in_specs=[_row_spec(d), _full_spec((d, d), 2),
                pl.BlockSpec((NC, BLK), lambda i: (0, i))],
      out_specs=[_row_spec(d), _vec_spec()],
      out_shape=[jax.ShapeDtypeStruct((n_pad, d), jnp.float32),
                 jax.ShapeDtypeStruct((n_pad,), jnp.float32)],
  )
  y1, dinv = k1(x_pad, W1, dparts)

  agg = _agg_kernel(n_pad, d, cpt)
  parts1 = agg(y1, src, dst, zeros_2d)

  k2 = pl.pallas_call(
      _k2_body,
      grid=(grid,),
      in_specs=[_parts_spec(d), _row_spec(d), _vec_spec(),
                _full_spec((d, d), 2), _full_spec((1, d), 2),
                _full_spec((1, d), 2)],
      out_specs=_row_spec(d),
      out_shape=jax.ShapeDtypeStruct((n_pad, d), jnp.float32),
  )
  y2 = k2(parts1, y1, dinv, W2, b1r, a1r)

  parts2 = agg(y2, src, dst, zeros_2d)

  k3 = pl.pallas_call(
      _k3_body,
      grid=(grid,),
      in_specs=[_parts_spec(d), _row_spec(d), _vec_spec(),
                _full_spec((1, d), 2), _full_spec((1, d), 2)],
      out_specs=_row_spec(d),
      out_shape=jax.ShapeDtypeStruct((n_pad, d), jnp.float32),
  )
  z = k3(parts2, y2, dinv, b2r, a2r)
  return z[:n]
